# baseline (speedup 1.0000x reference)
import functools

import jax
import jax.numpy as jnp
from jax.experimental import pallas as pl
from jax.experimental.pallas import tpu as pltpu

_INV_SQRT2 = 0.7071067811865476


def _gelu(hblk):
    return hblk * 0.5 * (1.0 + jax.lax.erf(hblk * _INV_SQRT2))


def _router_kernel(*refs, nh, e):
    tok_ref, gamma_ref, beta_ref = refs[0], refs[1], refs[2]
    w1_refs = refs[3:3 + nh]
    b1_refs = refs[3 + nh:3 + 2 * nh]
    w2_ref = refs[3 + 2 * nh]
    b2_ref = refs[3 + 2 * nh + 1]
    logits_ref, se_ref, ew_ref = refs[3 + 2 * nh + 2:3 + 2 * nh + 5]
    g_ref = refs[3 + 2 * nh + 5]

    x = tok_ref[...]
    mu = jnp.mean(x, axis=-1, keepdims=True)
    var = jnp.mean(x * x, axis=-1, keepdims=True) - mu * mu
    k = jax.lax.rsqrt(var + 1e-5)
    xn32 = (x - mu) * (k * gamma_ref[...]) + beta_ref[...]
    xn = xn32.astype(jnp.bfloat16)

    bh = w1_refs[0].shape[1]
    hprev = None
    for j in range(nh):
        hj = jnp.dot(xn, w1_refs[j][...],
                     preferred_element_type=jnp.float32) + b1_refs[j][...]
        if hprev is not None:
            g_ref[:, (j - 1) * bh:j * bh] = _gelu(hprev).astype(jnp.bfloat16)
        hprev = hj
    g_ref[:, (nh - 1) * bh:nh * bh] = _gelu(hprev).astype(jnp.bfloat16)
    logits = jnp.dot(g_ref[...], w2_ref[...],
                     preferred_element_type=jnp.float32) + b2_ref[...]

    logits_ref[...] = logits
    col = jax.lax.broadcasted_iota(jnp.int32, logits.shape, 1)
    m1 = jnp.max(logits, axis=1, keepdims=True)
    i1 = jnp.min(jnp.where(logits == m1, col, e), axis=1, keepdims=True)
    masked = jnp.where(col == i1, -jnp.inf, logits)
    m2 = jnp.max(masked, axis=1, keepdims=True)
    i2 = jnp.min(jnp.where(masked == m2, col, e), axis=1, keepdims=True)
    t = jnp.exp(m2 - m1)
    s = 1.0 + t
    wa = 1.0 / s
    wb = t / s
    ew_ref[...] = jnp.where(col == i1, wa, jnp.where(col == i2, wb, 0.0))
    se_ref[...] = jnp.concatenate([i1, i2], axis=1)


def kernel(tokens, gamma, beta, W1, b1, W2, b2):
    n, d = tokens.shape
    h = W1.shape[1]
    e = W2.shape[1]
    bm = min(256, n)
    bh = min(512, h)
    nh = h // bh
    grid = (n // bm,)

    def _const2(i, j):
        return lambda m: (i, j)

    in_specs = [
        pl.BlockSpec((bm, d), lambda m: (m, 0)),
        pl.BlockSpec((1, d), lambda m: (0, 0)),
        pl.BlockSpec((1, d), lambda m: (0, 0)),
    ]
    in_specs += [pl.BlockSpec((d, bh), _const2(0, j)) for j in range(nh)]
    in_specs += [pl.BlockSpec((1, bh), _const2(0, j)) for j in range(nh)]
    in_specs += [pl.BlockSpec((h, e), lambda m: (0, 0))]
    in_specs += [pl.BlockSpec((1, e), lambda m: (0, 0))]

    w1b = W1.astype(jnp.bfloat16)
    w2b = W2.astype(jnp.bfloat16)
    b1r = b1.reshape(1, h)

    out = pl.pallas_call(
        functools.partial(_router_kernel, nh=nh, e=e),
        grid=grid,
        compiler_params=pltpu.CompilerParams(
            vmem_limit_bytes=63 * 1024 * 1024),
        in_specs=in_specs,
        out_specs=[
            pl.BlockSpec((bm, e), lambda m: (m, 0)),
            pl.BlockSpec((bm, 2), lambda m: (m, 0)),
            pl.BlockSpec((bm, e), lambda m: (m, 0)),
        ],
        out_shape=[
            jax.ShapeDtypeStruct((n, e), jnp.float32),
            jax.ShapeDtypeStruct((n, 2), jnp.int32),
            jax.ShapeDtypeStruct((n, e), jnp.float32),
        ],
        scratch_shapes=[pltpu.VMEM((bm, h), jnp.bfloat16)],
    )(tokens, gamma.reshape(1, d), beta.reshape(1, d),
      *[w1b] * nh, *[b1r] * nh, w2b, b2.reshape(1, e))
    return (out[0], out[1], out[2])


# speedup vs baseline: 1.0332x; 1.0332x over previous
import functools

import jax
import jax.numpy as jnp
from jax.experimental import pallas as pl
from jax.experimental.pallas import tpu as pltpu

_SQRT2 = 1.4142135623730951


def _gelu(hblk):
    return hblk * (jax.lax.erf(hblk / _SQRT2) + 1.0) * 0.5


def _router_kernel(*refs, nh, e):
    tok_ref, gamma_ref, beta_ref = refs[0], refs[1], refs[2]
    w1_refs = refs[3:3 + nh]
    b1_refs = refs[3 + nh:3 + 2 * nh]
    w2_ref = refs[3 + 2 * nh]
    b2_ref = refs[3 + 2 * nh + 1]
    logits_ref, se_ref, ew_ref = refs[3 + 2 * nh + 2:3 + 2 * nh + 5]
    g_ref = refs[3 + 2 * nh + 5]

    x = tok_ref[...]
    mu = jnp.mean(x, axis=-1, keepdims=True)
    xc = x - mu
    var = jnp.mean(xc * xc, axis=-1, keepdims=True)
    rstd = 1.0 / jnp.sqrt(var + 1e-5)
    xn32 = xc * rstd * gamma_ref[...] + beta_ref[...]
    xn = xn32.astype(jnp.bfloat16)

    bh = w1_refs[0].shape[1]
    hprev = None
    for j in range(nh):
        hj = jnp.dot(xn, w1_refs[j][...],
                     preferred_element_type=jnp.float32) + b1_refs[j][...]
        if hprev is not None:
            g_ref[:, (j - 1) * bh:j * bh] = _gelu(hprev).astype(jnp.bfloat16)
        hprev = hj
    g_ref[:, (nh - 1) * bh:nh * bh] = _gelu(hprev).astype(jnp.bfloat16)
    logits = jnp.dot(g_ref[...], w2_ref[...],
                     preferred_element_type=jnp.float32) + b2_ref[...]

    logits_ref[...] = logits
    col = jax.lax.broadcasted_iota(jnp.int32, logits.shape, 1)
    m1 = jnp.max(logits, axis=1, keepdims=True)
    i1 = jnp.min(jnp.where(logits == m1, col, e), axis=1, keepdims=True)
    masked = jnp.where(col == i1, -jnp.inf, logits)
    m2 = jnp.max(masked, axis=1, keepdims=True)
    i2 = jnp.min(jnp.where(masked == m2, col, e), axis=1, keepdims=True)
    t = jnp.exp(m2 - m1)
    s = 1.0 + t
    wa = 1.0 / s
    wb = t / s
    ew_ref[...] = jnp.where(col == i1, wa, jnp.where(col == i2, wb, 0.0))
    se_ref[...] = jnp.concatenate([i1, i2], axis=1)


def kernel(tokens, gamma, beta, W1, b1, W2, b2):
    n, d = tokens.shape
    h = W1.shape[1]
    e = W2.shape[1]
    bm = min(512, n)
    bh = min(512, h)
    nh = h // bh
    grid = (n // bm,)

    def _const2(i, j):
        return lambda m: (i, j)

    in_specs = [
        pl.BlockSpec((bm, d), lambda m: (m, 0)),
        pl.BlockSpec((1, d), lambda m: (0, 0)),
        pl.BlockSpec((1, d), lambda m: (0, 0)),
    ]
    in_specs += [pl.BlockSpec((d, bh), _const2(0, j)) for j in range(nh)]
    in_specs += [pl.BlockSpec((1, bh), _const2(0, j)) for j in range(nh)]
    in_specs += [pl.BlockSpec((h, e), lambda m: (0, 0))]
    in_specs += [pl.BlockSpec((1, e), lambda m: (0, 0))]

    w1b = W1.astype(jnp.bfloat16)
    w2b = W2.astype(jnp.bfloat16)
    b1r = b1.reshape(1, h)

    out = pl.pallas_call(
        functools.partial(_router_kernel, nh=nh, e=e),
        grid=grid,
        compiler_params=pltpu.CompilerParams(
            vmem_limit_bytes=63 * 1024 * 1024),
        in_specs=in_specs,
        out_specs=[
            pl.BlockSpec((bm, e), lambda m: (m, 0)),
            pl.BlockSpec((bm, 2), lambda m: (m, 0)),
            pl.BlockSpec((bm, e), lambda m: (m, 0)),
        ],
        out_shape=[
            jax.ShapeDtypeStruct((n, e), jnp.float32),
            jax.ShapeDtypeStruct((n, 2), jnp.int32),
            jax.ShapeDtypeStruct((n, e), jnp.float32),
        ],
        scratch_shapes=[pltpu.VMEM((bm, h), jnp.bfloat16)],
    )(tokens, gamma.reshape(1, d), beta.reshape(1, d),
      *[w1b] * nh, *[b1r] * nh, w2b, b2.reshape(1, e))
    return (out[0], out[1], out[2])
